# baseline (speedup 1.0000x reference)
import functools

import jax
import jax.numpy as jnp
from jax import lax
from jax.experimental import pallas as pl
from jax.experimental.pallas import tpu as pltpu
from jax.experimental.pallas import tpu_sc as plsc

C = 16
S3 = 8
NW = 32
CH = 1000


def _make_sc_upsample(n):
    nchunks = n // CH
    mesh = plsc.VectorSubcoreMesh(core_axis_name="c", subcore_axis_name="s")

    @functools.partial(
        pl.kernel,
        mesh=mesh,
        out_type=[
            jax.ShapeDtypeStruct((n, S3, C), jnp.float32),
            jax.ShapeDtypeStruct((n * S3 * 3,), jnp.int32),
        ],
        scratch_types=[
            pltpu.VMEM((CH, C), jnp.float32),
            pltpu.VMEM((CH * 3,), jnp.int32),
            pltpu.VMEM((CH * S3 * 3,), jnp.int32),
            pltpu.SemaphoreType.DMA,
        ],
        compiler_params=pltpu.CompilerParams(
            needs_layout_passes=False, use_tc_tiling_on_sc=False),
    )
    def sc_upsample(data_hbm, ijk_hbm, out_data_hbm, out_ijk_hbm,
                    dbuf, ibuf, obuf, sem):
        wid = lax.axis_index("s") * 2 + lax.axis_index("c")

        lanes = lax.iota(jnp.int32, 16)
        gidx = []
        offv = []
        for v in range(3):
            p = lanes + v * 16
            i_rel = p // 24
            c = p % 3
            j = (p % 24) // 3
            gidx.append(i_rel * 3 + c)
            offv.append(
                jnp.where(c == 0, (j >> 2) & 1,
                          jnp.where(c == 1, (j >> 1) & 1, j & 1)))

        def chunk_body(i, carry):
            k = wid + i * NW

            @pl.when(k < nchunks)
            def _():
                r0 = k * CH
                pltpu.sync_copy(data_hbm.at[pl.ds(r0, CH)], dbuf)
                cps = [
                    pltpu.async_copy(
                        dbuf, out_data_hbm.at[pl.ds(r0, CH), j], sem)
                    for j in range(S3)
                ]
                pltpu.sync_copy(ijk_hbm.at[pl.ds(r0 * 3, CH * 3)], ibuf)
                def g_body(g, carry2):
                    base = g * 6
                    ob = g * 48
                    for v in range(3):
                        x = plsc.load_gather(ibuf, [gidx[v] + base])
                        obuf[pl.ds(ob + v * 16, 16)] = x * 2 + offv[v]
                    return carry2

                lax.fori_loop(0, CH // 2, g_body, 0, unroll=4)
                pltpu.sync_copy(obuf,
                                out_ijk_hbm.at[pl.ds(r0 * 24, CH * 24)])
                for cp in cps:
                    cp.wait()

            return carry

        lax.fori_loop(0, (nchunks + NW - 1) // NW, chunk_body, 0)

    return sc_upsample


def kernel(coarse_data, coarse_ijk):
    n = coarse_data.shape[0]
    fn = _make_sc_upsample(n)
    fine_data, fine_ijk = fn(coarse_data, coarse_ijk.reshape(n * 3))
    return fine_data.reshape(n * S3, C), fine_ijk.reshape(n * S3, 3)


# speedup vs baseline: 5.6534x; 5.6534x over previous
import functools

import jax
import jax.numpy as jnp
from jax import lax
from jax.experimental import pallas as pl
from jax.experimental.pallas import tpu as pltpu
from jax.experimental.pallas import tpu_sc as plsc

C = 16
S3 = 8
NW = 32
CIN = 256


def _expansion_patterns():
    lanes = lax.iota(jnp.int32, 16)
    l8 = lanes // 8
    j = lanes % 8
    offc = [(j >> 2) & 1, (j >> 1) & 1, j & 1]
    return l8, offc


def _expand(src_d, src_i, dst_d, dst_i, ncols, l8, offc, rowv):
    def g_body(g, carry):
        gbase = g * 16
        for kk in range(S3):
            idx = gbase + (kk * 2 + l8)
            ob = g * 128 + kk * 16
            for ch in range(C):
                x = plsc.load_gather(src_d, [rowv[ch], idx])
                dst_d[ch, pl.ds(ob, 16)] = x
            for r in range(3):
                y = plsc.load_gather(src_i, [rowv[r], idx])
                dst_i[r, pl.ds(ob, 16)] = y * 2 + offc[r]
        return carry

    lax.fori_loop(0, ncols // 16, g_body, 0)


def _make_sc_upsample(n):
    naligned = (n // 128) * 128
    tail = n - naligned
    assert naligned % CIN == 0
    nfull = naligned // CIN
    nchunks = nfull + (1 if tail else 0)
    mesh = plsc.VectorSubcoreMesh(core_axis_name="c", subcore_axis_name="s")

    scratch = [
        pltpu.VMEM((C, CIN), jnp.float32),
        pltpu.VMEM((3, CIN), jnp.int32),
        pltpu.VMEM((C, CIN * S3), jnp.float32),
        pltpu.VMEM((3, CIN * S3), jnp.int32),
        pltpu.SemaphoreType.DMA,
    ]
    if tail:
        scratch += [
            pltpu.VMEM((C, tail), jnp.float32),
            pltpu.VMEM((3, tail), jnp.int32),
            pltpu.VMEM((C, tail * S3), jnp.float32),
            pltpu.VMEM((3, tail * S3), jnp.int32),
        ]

    @functools.partial(
        pl.kernel,
        mesh=mesh,
        out_type=[
            jax.ShapeDtypeStruct((C, n * S3), jnp.float32),
            jax.ShapeDtypeStruct((3, n * S3), jnp.int32),
        ],
        scratch_types=scratch,
        compiler_params=pltpu.CompilerParams(needs_layout_passes=False),
    )
    def sc_upsample(data_hbm, ijk_hbm, dtail_hbm, itail_hbm,
                    out_data_hbm, out_ijk_hbm,
                    dbuf, ibuf, odbuf, oibuf, sem, *tailbufs):
        wid = lax.axis_index("s") * 2 + lax.axis_index("c")
        l8, offc = _expansion_patterns()
        rowv = [jnp.full((16,), r, jnp.int32) for r in range(C)]

        def chunk_body(i, carry):
            k = wid + i * NW

            @pl.when(k < nfull)
            def _():
                c0 = k * CIN
                pltpu.sync_copy(data_hbm.at[:, pl.ds(c0, CIN)], dbuf)
                pltpu.sync_copy(ijk_hbm.at[:, pl.ds(c0, CIN)], ibuf)
                _expand(dbuf, ibuf, odbuf, oibuf, CIN, l8, offc, rowv)
                cpd = pltpu.async_copy(
                    odbuf, out_data_hbm.at[:, pl.ds(c0 * S3, CIN * S3)], sem)
                cpi = pltpu.async_copy(
                    oibuf, out_ijk_hbm.at[:, pl.ds(c0 * S3, CIN * S3)], sem)
                cpd.wait()
                cpi.wait()

            if tail:
                tdbuf, tibuf, todbuf, toibuf = tailbufs

                @pl.when(k == nfull)
                def _():
                    pltpu.sync_copy(dtail_hbm, tdbuf)
                    pltpu.sync_copy(itail_hbm, tibuf)
                    _expand(tdbuf, tibuf, todbuf, toibuf, tail, l8, offc,
                            rowv)
                    cpd = pltpu.async_copy(
                        todbuf,
                        out_data_hbm.at[:, pl.ds(naligned * S3, tail * S3)],
                        sem)
                    cpi = pltpu.async_copy(
                        toibuf,
                        out_ijk_hbm.at[:, pl.ds(naligned * S3, tail * S3)],
                        sem)
                    cpd.wait()
                    cpi.wait()

            return carry

        lax.fori_loop(0, (nchunks + NW - 1) // NW, chunk_body, 0)

    return sc_upsample


def kernel(coarse_data, coarse_ijk):
    n = coarse_data.shape[0]
    naligned = (n // 128) * 128
    dt = coarse_data.T
    it = coarse_ijk.T
    dtail = lax.slice(dt, (0, naligned), (C, n))
    itail = lax.slice(it, (0, naligned), (3, n))
    fn = _make_sc_upsample(n)
    fine_data_t, fine_ijk_t = fn(dt, it, dtail, itail)
    return fine_data_t.T, fine_ijk_t.T


# speedup vs baseline: 6.6514x; 1.1765x over previous
import functools

import jax
import jax.numpy as jnp
from jax import lax
from jax.experimental import pallas as pl
from jax.experimental.pallas import tpu as pltpu
from jax.experimental.pallas import tpu_sc as plsc

C = 16
S3 = 8
NW = 32
CIN = 256


def _expansion_patterns():
    lanes = lax.iota(jnp.int32, 16)
    l8 = lanes // 8
    j = lanes % 8
    offc = [(j >> 2) & 1, (j >> 1) & 1, j & 1]
    rowv = [jnp.full((16,), r, jnp.int32) for r in range(C)]
    return l8, offc, rowv


def _expand(src_d, src_i, dst_d, dst_i, ncols, pats):
    l8, offc, rowv = pats

    def g_body(g, carry):
        gbase = g * 16
        for kk in range(S3):
            idx = gbase + (kk * 2 + l8)
            ob = g * 128 + kk * 16
            for ch in range(C):
                x = plsc.load_gather(src_d, [rowv[ch], idx])
                dst_d[ch, pl.ds(ob, 16)] = x
            for r in range(3):
                y = plsc.load_gather(src_i, [rowv[r], idx])
                dst_i[r, pl.ds(ob, 16)] = y * 2 + offc[r]
        return carry

    lax.fori_loop(0, ncols // 16, g_body, 0)


def _make_sc_upsample(n):
    naligned = (n // 128) * 128
    tail = n - naligned
    assert naligned % CIN == 0
    nfull = naligned // CIN
    mesh = plsc.VectorSubcoreMesh(core_axis_name="c", subcore_axis_name="s")

    scratch = [
        pltpu.VMEM((C, CIN), jnp.float32),
        pltpu.VMEM((3, CIN), jnp.int32),
        pltpu.VMEM((C, CIN * S3), jnp.float32),
        pltpu.VMEM((C, CIN * S3), jnp.float32),
        pltpu.VMEM((3, CIN * S3), jnp.int32),
        pltpu.VMEM((3, CIN * S3), jnp.int32),
        pltpu.SemaphoreType.DMA((6,)),
    ]
    if tail:
        scratch += [
            pltpu.VMEM((C, tail), jnp.float32),
            pltpu.VMEM((3, tail), jnp.int32),
            pltpu.VMEM((C, tail * S3), jnp.float32),
            pltpu.VMEM((3, tail * S3), jnp.int32),
        ]

    @functools.partial(
        pl.kernel,
        mesh=mesh,
        out_type=[
            jax.ShapeDtypeStruct((C, n * S3), jnp.float32),
            jax.ShapeDtypeStruct((3, n * S3), jnp.int32),
        ],
        scratch_types=scratch,
        compiler_params=pltpu.CompilerParams(needs_layout_passes=False),
    )
    def sc_upsample(data_hbm, ijk_hbm, dtail_hbm, itail_hbm,
                    out_data_hbm, out_ijk_hbm,
                    dbuf, ibuf, odbuf0, odbuf1, oibuf0, oibuf1, sem,
                    *tailbufs):
        wid = lax.axis_index("s") * 2 + lax.axis_index("c")
        pats = _expansion_patterns()
        odbufs = (odbuf0, odbuf1)
        oibufs = (oibuf0, oibuf1)

        def out_slices(c0):
            return (out_data_hbm.at[:, pl.ds(c0 * S3, CIN * S3)],
                    out_ijk_hbm.at[:, pl.ds(c0 * S3, CIN * S3)])

        def do_main(i, h):
            k = wid + i * NW

            @pl.when(k < nfull)
            def _():
                c0 = k * CIN
                cin_d = pltpu.async_copy(
                    data_hbm.at[:, pl.ds(c0, CIN)], dbuf, sem.at[0])
                cin_i = pltpu.async_copy(
                    ijk_hbm.at[:, pl.ds(c0, CIN)], ibuf, sem.at[1])
                cin_d.wait()
                cin_i.wait()
                od_hbm, oi_hbm = out_slices(c0)

                @pl.when(i >= 2)
                def _():
                    pltpu.make_async_copy(
                        odbufs[h], od_hbm, sem.at[2 + h]).wait()
                    pltpu.make_async_copy(
                        oibufs[h], oi_hbm, sem.at[4 + h]).wait()

                _expand(dbuf, ibuf, odbufs[h], oibufs[h], CIN, pats)
                pltpu.async_copy(odbufs[h], od_hbm, sem.at[2 + h])
                pltpu.async_copy(oibufs[h], oi_hbm, sem.at[4 + h])

        def do_tail(i):
            if not tail:
                return
            tdbuf, tibuf, todbuf, toibuf = tailbufs
            k = wid + i * NW

            @pl.when(k == nfull)
            def _():
                pltpu.sync_copy(dtail_hbm, tdbuf)
                pltpu.sync_copy(itail_hbm, tibuf)
                _expand(tdbuf, tibuf, todbuf, toibuf, tail, pats)
                pltpu.sync_copy(
                    todbuf,
                    out_data_hbm.at[:, pl.ds(naligned * S3, tail * S3)])
                pltpu.sync_copy(
                    toibuf,
                    out_ijk_hbm.at[:, pl.ds(naligned * S3, tail * S3)])

        nchunks = nfull + (1 if tail else 0)
        nit = (nchunks + NW - 1) // NW

        def pair_body(i2, carry):
            for h in (0, 1):
                i = 2 * i2 + h
                do_main(i, h)
                do_tail(i)
            return carry

        lax.fori_loop(0, (nit + 1) // 2, pair_body, 0)

        od_hbm, oi_hbm = out_slices(0)
        for h in (0, 1):
            pltpu.make_async_copy(odbufs[h], od_hbm, sem.at[2 + h]).wait()
            pltpu.make_async_copy(oibufs[h], oi_hbm, sem.at[4 + h]).wait()

    return sc_upsample


def kernel(coarse_data, coarse_ijk):
    n = coarse_data.shape[0]
    naligned = (n // 128) * 128
    dt = coarse_data.T
    it = coarse_ijk.T
    dtail = lax.slice(dt, (0, naligned), (C, n))
    itail = lax.slice(it, (0, naligned), (3, n))
    fn = _make_sc_upsample(n)
    fine_data_t, fine_ijk_t = fn(dt, it, dtail, itail)
    return fine_data_t.T, fine_ijk_t.T


# speedup vs baseline: 14.9141x; 2.2423x over previous
import functools

import jax
import jax.numpy as jnp
from jax import lax
from jax.experimental import pallas as pl
from jax.experimental.pallas import tpu as pltpu
from jax.experimental.pallas import tpu_sc as plsc

C = 16
S3 = 8
NW = 32
CIN = 256


def _expansion_patterns():
    lanes = lax.iota(jnp.int32, 16)
    l8 = lanes // 8
    j = lanes % 8
    offc = [(j >> 2) & 1, (j >> 1) & 1, j & 1]
    rowv = [jnp.full((16,), r, jnp.int32) for r in range(C)]
    return l8, offc, rowv


def _expand(src_d, src_i, dst_d, dst_i, ncols, pats):
    l8, offc, rowv = pats

    def g_body(g, carry):
        gbase = g * 16
        for kk in range(S3):
            idx = gbase + (kk * 2 + l8)
            ob = g * 128 + kk * 16
            xs = [plsc.load_gather(src_d, [rowv[ch], idx])
                  for ch in range(C)]
            ys = [plsc.load_gather(src_i, [rowv[r], idx]) * 2 + offc[r]
                  for r in range(3)]
            for ch in range(C):
                dst_d[ch, pl.ds(ob, 16)] = xs[ch]
            for r in range(3):
                dst_i[r, pl.ds(ob, 16)] = ys[r]
        return carry

    lax.fori_loop(0, ncols // 16, g_body, 0)


def _make_sc_upsample(n):
    naligned = (n // 128) * 128
    tail = n - naligned
    assert naligned % CIN == 0
    nfull = naligned // CIN
    mesh = plsc.VectorSubcoreMesh(core_axis_name="c", subcore_axis_name="s")

    scratch = [
        pltpu.VMEM((C, CIN), jnp.float32),
        pltpu.VMEM((3, CIN), jnp.int32),
        pltpu.VMEM((C, CIN * S3), jnp.float32),
        pltpu.VMEM((C, CIN * S3), jnp.float32),
        pltpu.VMEM((3, CIN * S3), jnp.int32),
        pltpu.VMEM((3, CIN * S3), jnp.int32),
        pltpu.SemaphoreType.DMA((6,)),
    ]
    if tail:
        scratch += [
            pltpu.VMEM((C, tail), jnp.float32),
            pltpu.VMEM((3, tail), jnp.int32),
            pltpu.VMEM((C, tail * S3), jnp.float32),
            pltpu.VMEM((3, tail * S3), jnp.int32),
        ]

    @functools.partial(
        pl.kernel,
        mesh=mesh,
        out_type=[
            jax.ShapeDtypeStruct((C, n * S3), jnp.float32),
            jax.ShapeDtypeStruct((3, n * S3), jnp.int32),
        ],
        scratch_types=scratch,
        compiler_params=pltpu.CompilerParams(needs_layout_passes=False),
    )
    def sc_upsample(data_hbm, ijk_hbm, dtail_hbm, itail_hbm,
                    out_data_hbm, out_ijk_hbm,
                    dbuf, ibuf, odbuf0, odbuf1, oibuf0, oibuf1, sem,
                    *tailbufs):
        wid = lax.axis_index("s") * 2 + lax.axis_index("c")
        pats = _expansion_patterns()
        odbufs = (odbuf0, odbuf1)
        oibufs = (oibuf0, oibuf1)

        def out_slices(c0):
            return (out_data_hbm.at[:, pl.ds(c0 * S3, CIN * S3)],
                    out_ijk_hbm.at[:, pl.ds(c0 * S3, CIN * S3)])

        def do_main(i, h):
            k = wid + i * NW

            @pl.when(k < nfull)
            def _():
                c0 = k * CIN
                cin_d = pltpu.async_copy(
                    data_hbm.at[:, pl.ds(c0, CIN)], dbuf, sem.at[0])
                cin_i = pltpu.async_copy(
                    ijk_hbm.at[:, pl.ds(c0, CIN)], ibuf, sem.at[1])
                cin_d.wait()
                cin_i.wait()
                od_hbm, oi_hbm = out_slices(c0)

                @pl.when(i >= 2)
                def _():
                    pltpu.make_async_copy(
                        odbufs[h], od_hbm, sem.at[2 + h]).wait()
                    pltpu.make_async_copy(
                        oibufs[h], oi_hbm, sem.at[4 + h]).wait()

                _expand(dbuf, ibuf, odbufs[h], oibufs[h], CIN, pats)
                pltpu.async_copy(odbufs[h], od_hbm, sem.at[2 + h])
                pltpu.async_copy(oibufs[h], oi_hbm, sem.at[4 + h])

        def do_tail(i):
            if not tail:
                return
            tdbuf, tibuf, todbuf, toibuf = tailbufs
            k = wid + i * NW

            @pl.when(k == nfull)
            def _():
                pltpu.sync_copy(dtail_hbm, tdbuf)
                pltpu.sync_copy(itail_hbm, tibuf)
                _expand(tdbuf, tibuf, todbuf, toibuf, tail, pats)
                pltpu.sync_copy(
                    todbuf,
                    out_data_hbm.at[:, pl.ds(naligned * S3, tail * S3)])
                pltpu.sync_copy(
                    toibuf,
                    out_ijk_hbm.at[:, pl.ds(naligned * S3, tail * S3)])

        nchunks = nfull + (1 if tail else 0)
        nit = (nchunks + NW - 1) // NW

        def pair_body(i2, carry):
            for h in (0, 1):
                i = 2 * i2 + h
                do_main(i, h)
                do_tail(i)
            return carry

        lax.fori_loop(0, (nit + 1) // 2, pair_body, 0)

        od_hbm, oi_hbm = out_slices(0)
        for h in (0, 1):
            pltpu.make_async_copy(odbufs[h], od_hbm, sem.at[2 + h]).wait()
            pltpu.make_async_copy(oibufs[h], oi_hbm, sem.at[4 + h]).wait()

    return sc_upsample


def kernel(coarse_data, coarse_ijk):
    n = coarse_data.shape[0]
    naligned = (n // 128) * 128
    dt = coarse_data.T
    it = coarse_ijk.T
    dtail = lax.slice(dt, (0, naligned), (C, n))
    itail = lax.slice(it, (0, naligned), (3, n))
    fn = _make_sc_upsample(n)
    fine_data_t, fine_ijk_t = fn(dt, it, dtail, itail)
    return fine_data_t.T, fine_ijk_t.T
